# baseline (speedup 1.0000x reference)
import functools

import jax
import jax.numpy as jnp
from jax import lax
from jax.experimental import pallas as pl
from jax.experimental.pallas import tpu as pltpu
from jax.experimental.pallas import tpu_sc as plsc

E = 160000
D = 128
TD = 32
N = 10000
NQP = 10240
BLK = 2000
GB = 128
NBLK = E // GB
NC, NS = 2, 16
NW = NC * NS
NP = 10240
NPT = NP // NS

_sc_mesh = plsc.VectorSubcoreMesh(
    core_axis_name="c", subcore_axis_name="s", num_cores=NC, num_subcores=NS)


def _share(total, workers, w):
    return total // workers + jnp.where(w < total % workers, 1, 0)


def _qr_body(qrel_h, rela_h, qrt_o, idx_v, rows_v):
    cid = lax.axis_index("c")
    sid = lax.axis_index("s")
    wid = sid * NC + cid

    def body(i, carry):
        base = (wid + i * NW) * GB
        pltpu.sync_copy(qrel_h.at[pl.ds(base, GB)], idx_v)
        pltpu.sync_copy(rela_h.at[idx_v], rows_v)
        pltpu.sync_copy(rows_v, qrt_o.at[pl.ds(base, GB)])
        return carry

    lax.fori_loop(0, _share(NQP // GB, NW, wid), body, 0)


def _qr_stage(q_rel_pad, rela_embed):
    return pl.kernel(
        _qr_body,
        out_type=jax.ShapeDtypeStruct((NQP, D), jnp.float32),
        mesh=_sc_mesh,
        scratch_types=[pltpu.VMEM((GB,), jnp.int32),
                       pltpu.VMEM((GB, D), jnp.float32)],
    )(q_rel_pad, rela_embed)


def _gather_body(sub_h, e2_h, ridx_h, hidden_h, rela_h, qrt_h,
                 hs_o, rel_o, hqr_o,
                 sub_v, e2_v, ridx_v,
                 hs_v, rel_v, hqr_v, s0, s1, s3):
    cid = lax.axis_index("c")
    sid = lax.axis_index("s")
    wid = sid * NC + cid

    def body(i, carry):
        base = (wid + i * NW) * GB
        pltpu.sync_copy(sub_h.at[pl.ds(base, GB)], sub_v)
        pltpu.sync_copy(e2_h.at[pl.ds(base, GB)], e2_v)
        pltpu.sync_copy(ridx_h.at[pl.ds(base, GB)], ridx_v)
        c0 = pltpu.async_copy(hidden_h.at[sub_v], hs_v, s0)
        c1 = pltpu.async_copy(rela_h.at[e2_v], rel_v, s1)
        c3 = pltpu.async_copy(qrt_h.at[ridx_v], hqr_v, s3)
        c0.wait()
        pltpu.sync_copy(hs_v, hs_o.at[pl.ds(base, GB)])
        c1.wait()
        pltpu.sync_copy(rel_v, rel_o.at[pl.ds(base, GB)])
        c3.wait()
        pltpu.sync_copy(hqr_v, hqr_o.at[pl.ds(base, GB)])
        return carry

    lax.fori_loop(0, _share(NBLK, NW, wid), body, 0)


def _gather_stage(sub, e2, r_idx, hidden, rela_embed, qr_table):
    f32 = jnp.float32
    i32 = jnp.int32
    return pl.kernel(
        _gather_body,
        out_type=[jax.ShapeDtypeStruct((E, D), f32),
                  jax.ShapeDtypeStruct((E, D), f32),
                  jax.ShapeDtypeStruct((E, D), f32)],
        mesh=_sc_mesh,
        scratch_types=[pltpu.VMEM((GB,), i32), pltpu.VMEM((GB,), i32),
                       pltpu.VMEM((GB,), i32),
                       pltpu.VMEM((GB, D), f32), pltpu.VMEM((GB, D), f32),
                       pltpu.VMEM((GB, D), f32),
                       pltpu.SemaphoreType.DMA, pltpu.SemaphoreType.DMA,
                       pltpu.SemaphoreType.DMA],
    )(sub, e2, r_idx, hidden, rela_embed, qr_table)


def _dense_body(rel_r, e6_r, hs_r, hqr_r, head_r, tail_r,
                div_r, W1a_r, W1s_r, W1c_r, b1_r, W2_r, b2_r,
                Wg1_r, Wg2_r, Wg3_r, Wg4_r, Wg5_r, bg_r,
                Wt1_r, Wt2_r, bt_r,
                Ws_r, Wr_r, Wq1_r, Wq2_r, Wq3_r, bqr_r, wa_r,
                up_r):
    lr = lambda x: jnp.where(x > 0, x, 0.01 * x)
    dot = lambda a, b: jnp.dot(a, b, preferred_element_type=jnp.float32)
    rel = rel_r[...]
    hs = hs_r[...]
    hqr = hqr_r[...]
    head = head_r[...]
    tail = tail_r[...]
    ang = e6_r[...].astype(jnp.float32) * div_r[...]
    h1 = lr(dot(rel, W1a_r[...]) + dot(jnp.sin(ang), W1s_r[...])
            + dot(jnp.cos(ang), W1c_r[...]) + b1_r[...])
    h2 = lr(dot(h1, W2_r[...]) + b2_r[...])
    hr = h2 + rel
    gin = (dot(hr, Wg1_r[...])
           + 0.25 * (dot(hqr, Wg2_r[...]) + dot(head, Wg3_r[...])
                     + dot(tail, Wg4_r[...]))
           + dot(hs, Wg5_r[...]) + bg_r[...])
    gates = jax.nn.sigmoid(gin)
    update = gates[:, :D]
    reset = gates[:, D:]
    cand = jnp.tanh(dot(hr, Wt1_r[...]) + dot(reset * hs, Wt2_r[...]) + bt_r[...])
    message = (1.0 - update) * hs + update * cand
    att = lr(dot(hs, Ws_r[...]) + dot(hr, Wr_r[...]) + dot(hqr, Wq1_r[...])
             + dot(head, Wq2_r[...]) + dot(tail, Wq3_r[...]) + bqr_r[...])
    alpha = dot(att, wa_r[...])
    up_r[...] = jax.nn.sigmoid(alpha) * message


def _dense_stage(rel, e6, hs, hqr, head, tail, weights):
    nblk = E // BLK
    row_spec = lambda w: pl.BlockSpec((BLK, w), lambda i: (i, 0))
    full = lambda a: pl.BlockSpec(a.shape, lambda i: (0,) * a.ndim)
    return pl.pallas_call(
        _dense_body,
        grid=(nblk,),
        in_specs=[row_spec(D), row_spec(1), row_spec(D), row_spec(D),
                  row_spec(D), row_spec(D)] + [full(w) for w in weights],
        out_specs=row_spec(D),
        out_shape=jax.ShapeDtypeStruct((E, D), jnp.float32),
    )(rel, e6, hs, hqr, head, tail, *weights)


def _scatter_body(obj_h, up_h, ones_h, zu_h,
                  pu_o, pd_o,
                  obj_v, up_v, acc):
    cid = lax.axis_index("c")
    sid = lax.axis_index("s")
    r0 = sid * NPT
    pltpu.sync_copy(zu_h.at[pl.ds(r0, NPT)], acc.at[pl.ds(r0, NPT)])
    plsc.subcore_barrier()

    @pl.when(cid == 0)
    def _up_core():
        def body(i, carry):
            base = (sid + i * NS) * GB
            pltpu.sync_copy(obj_h.at[pl.ds(base, GB)], obj_v)
            pltpu.sync_copy(up_h.at[pl.ds(base, GB)], up_v)
            pltpu.sync_copy(up_v, acc.at[obj_v], add=True)
            return carry

        lax.fori_loop(0, _share(NBLK, NS, sid), body, 0)

    @pl.when(cid == 1)
    def _deg_core():
        pltpu.sync_copy(ones_h, up_v)

        def body(i, carry):
            base = (sid + i * NS) * GB
            pltpu.sync_copy(obj_h.at[pl.ds(base, GB)], obj_v)
            pltpu.sync_copy(up_v, acc.at[obj_v], add=True)
            return carry

        lax.fori_loop(0, _share(NBLK, NS, sid), body, 0)

    plsc.subcore_barrier()

    @pl.when(cid == 0)
    def _out_up():
        pltpu.sync_copy(acc.at[pl.ds(r0, NPT)], pu_o.at[pl.ds(r0, NPT)])

    @pl.when(cid == 1)
    def _out_deg():
        pltpu.sync_copy(acc.at[pl.ds(r0, NPT)], pd_o.at[pl.ds(r0, NPT)])


def _scatter_stage(obj, up):
    f32 = jnp.float32
    ones = jnp.ones((GB, D), f32)
    zu = jnp.zeros((NP, D), f32)
    return pl.kernel(
        _scatter_body,
        out_type=[jax.ShapeDtypeStruct((NP, D), f32),
                  jax.ShapeDtypeStruct((NP, D), f32)],
        mesh=_sc_mesh,
        scratch_types=[pltpu.VMEM((GB,), jnp.int32),
                       pltpu.VMEM((GB, D), f32),
                       pltpu.VMEM_SHARED((NP, D), f32)],
    )(obj, up, ones, zu)


def _final_body(pu_r, pd_r, Wh_r, out_r):
    deg = pd_r[:, 0:1]
    agg = pu_r[...] / jnp.sqrt(deg + 0.0001)
    out_r[...] = jnp.dot(agg, Wh_r[...], preferred_element_type=jnp.float32)


def _final_stage(part_up, part_deg, Wh):
    rb = 2000
    return pl.pallas_call(
        _final_body,
        grid=(N // rb,),
        in_specs=[pl.BlockSpec((rb, D), lambda i: (i, 0)),
                  pl.BlockSpec((rb, D), lambda i: (i, 0)),
                  pl.BlockSpec(Wh.shape, lambda i: (0, 0))],
        out_specs=pl.BlockSpec((rb, D), lambda i: (i, 0)),
        out_shape=jax.ShapeDtypeStruct((N, D), jnp.float32),
    )(part_up, part_deg, Wh)


def kernel(q_sub, q_rel, hidden, edges, n_node, edge_head_rc_repr,
           edge_tail_rc_repr, query_head_rc_repr, rela_embed, time_pe,
           Ws, Wr, W1f, b1f, W2f, b2f, Wqr, bqr, w_alpha, Wg, bg,
           Wt, bt, Wh):
    sub = edges[:, 4]
    obj = edges[:, 5]
    r_idx = edges[:, 0]
    e2 = edges[:, 2]
    e6 = edges[:, 6]
    obj = obj + (jnp.asarray(n_node, dtype=obj.dtype) - N)

    q_rel_pad = jnp.pad(q_rel.astype(jnp.int32), (0, NQP - q_rel.shape[0]))
    qr_table = _qr_stage(q_rel_pad, rela_embed)

    hs, rel, hqr = _gather_stage(sub, e2, r_idx, hidden, rela_embed, qr_table)

    div = jnp.exp(jnp.arange(0, TD, 2, dtype=jnp.float32)
                  * -(jnp.log(jnp.float32(10000.0)) / TD)).reshape(1, TD // 2)
    W1b = W1f[D:]
    weights = (
        div, W1f[:D], W1b[0::2], W1b[1::2], b1f.reshape(1, D), W2f,
        b2f.reshape(1, D),
        Wg[:D], Wg[D:2 * D], Wg[2 * D:3 * D], Wg[3 * D:4 * D], Wg[4 * D:],
        bg.reshape(1, 2 * D),
        Wt[:D], Wt[D:], bt.reshape(1, D),
        Ws, Wr, Wqr[:D], Wqr[D:2 * D], Wqr[2 * D:], bqr.reshape(1, D),
        w_alpha,
    )
    up = _dense_stage(rel, e6.reshape(E, 1).astype(jnp.int32), hs, hqr,
                      edge_head_rc_repr, edge_tail_rc_repr, weights)

    part_up, part_deg = _scatter_stage(obj.astype(jnp.int32), up)

    return _final_stage(part_up, part_deg, Wh)


# speedup vs baseline: 1.1512x; 1.1512x over previous
import functools

import jax
import jax.numpy as jnp
from jax import lax
from jax.experimental import pallas as pl
from jax.experimental.pallas import tpu as pltpu
from jax.experimental.pallas import tpu_sc as plsc

E = 160000
D = 128
TD = 32
N = 10000
NQP = 10240
BLK = 2000
GB = 128
NBLK = E // GB
NC, NS = 2, 16
NW = NC * NS
NP = 10240
NPT = NP // NS

@functools.lru_cache(maxsize=1)
def _sc_mesh():
    return plsc.VectorSubcoreMesh(
        core_axis_name="c", subcore_axis_name="s",
        num_cores=NC, num_subcores=NS)


def _share(total, workers, w):
    return total // workers + jnp.where(w < total % workers, 1, 0)


def _qr_body(qrel_h, rela_h, qrt_o, idx_v, rows_v):
    cid = lax.axis_index("c")
    sid = lax.axis_index("s")
    wid = sid * NC + cid

    def body(i, carry):
        base = (wid + i * NW) * GB
        pltpu.sync_copy(qrel_h.at[pl.ds(base, GB)], idx_v)
        pltpu.sync_copy(rela_h.at[idx_v], rows_v)
        pltpu.sync_copy(rows_v, qrt_o.at[pl.ds(base, GB)])
        return carry

    lax.fori_loop(0, _share(NQP // GB, NW, wid), body, 0)


def _qr_stage(q_rel_pad, rela_embed):
    return pl.kernel(
        _qr_body,
        out_type=jax.ShapeDtypeStruct((NQP, D), jnp.float32),
        mesh=_sc_mesh(),
        scratch_types=[pltpu.VMEM((GB,), jnp.int32),
                       pltpu.VMEM((GB, D), jnp.float32)],
    )(q_rel_pad, rela_embed)


def _gather_body(sub_h, e2_h, e6_h, ridx_h, hidden_h, rela_h, tpw_h, qrt_h,
                 hs_o, rel_o, tpw_o, hqr_o,
                 sub_v, e2_v, e6_v, ridx_v,
                 hs_v, rel_v, tpw_v, hqr_v, s0, s1, s2, s3):
    cid = lax.axis_index("c")
    sid = lax.axis_index("s")
    wid = sid * NC + cid

    def body(i, carry):
        base = (wid + i * NW) * GB
        pltpu.sync_copy(sub_h.at[pl.ds(base, GB)], sub_v)
        pltpu.sync_copy(e2_h.at[pl.ds(base, GB)], e2_v)
        pltpu.sync_copy(e6_h.at[pl.ds(base, GB)], e6_v)
        pltpu.sync_copy(ridx_h.at[pl.ds(base, GB)], ridx_v)
        c0 = pltpu.async_copy(hidden_h.at[sub_v], hs_v, s0)
        c1 = pltpu.async_copy(rela_h.at[e2_v], rel_v, s1)
        c2 = pltpu.async_copy(tpw_h.at[e6_v], tpw_v, s2)
        c3 = pltpu.async_copy(qrt_h.at[ridx_v], hqr_v, s3)
        c0.wait()
        pltpu.sync_copy(hs_v, hs_o.at[pl.ds(base, GB)])
        c1.wait()
        pltpu.sync_copy(rel_v, rel_o.at[pl.ds(base, GB)])
        c2.wait()
        pltpu.sync_copy(tpw_v, tpw_o.at[pl.ds(base, GB)])
        c3.wait()
        pltpu.sync_copy(hqr_v, hqr_o.at[pl.ds(base, GB)])
        return carry

    lax.fori_loop(0, _share(NBLK, NW, wid), body, 0)


def _gather_stage(sub, e2, e6, r_idx, hidden, rela_embed, tpw, qr_table):
    f32 = jnp.float32
    i32 = jnp.int32
    return pl.kernel(
        _gather_body,
        out_type=[jax.ShapeDtypeStruct((E, D), f32),
                  jax.ShapeDtypeStruct((E, D), f32),
                  jax.ShapeDtypeStruct((E, D), f32),
                  jax.ShapeDtypeStruct((E, D), f32)],
        mesh=_sc_mesh(),
        scratch_types=[pltpu.VMEM((GB,), i32), pltpu.VMEM((GB,), i32),
                       pltpu.VMEM((GB,), i32), pltpu.VMEM((GB,), i32),
                       pltpu.VMEM((GB, D), f32), pltpu.VMEM((GB, D), f32),
                       pltpu.VMEM((GB, D), f32), pltpu.VMEM((GB, D), f32),
                       pltpu.SemaphoreType.DMA, pltpu.SemaphoreType.DMA,
                       pltpu.SemaphoreType.DMA, pltpu.SemaphoreType.DMA],
    )(sub, e2, e6, r_idx, hidden, rela_embed, tpw, qr_table)


def _tpw_body(tp_r, W1b_r, b1_r, out_r):
    out_r[...] = (jnp.dot(tp_r[...], W1b_r[...],
                          preferred_element_type=jnp.float32) + b1_r[...])


def _tpw_stage(time_pe, W1b, b1f):
    rb = 2000
    return pl.pallas_call(
        _tpw_body,
        grid=(N // rb,),
        in_specs=[pl.BlockSpec((rb, TD), lambda i: (i, 0)),
                  pl.BlockSpec((TD, D), lambda i: (0, 0)),
                  pl.BlockSpec((1, D), lambda i: (0, 0))],
        out_specs=pl.BlockSpec((rb, D), lambda i: (i, 0)),
        out_shape=jax.ShapeDtypeStruct((N, D), jnp.float32),
    )(time_pe, W1b, b1f)


def _dense_body(rel_r, tpw_r, hs_r, hqr_r, head_r, tail_r,
                W1a_r, W2_r, b2_r,
                Wg1_r, Wg2_r, Wg3_r, Wg4_r, Wg5_r, bg_r,
                Wt1_r, Wt2_r, bt_r,
                Ws_r, Wr_r, Wq1_r, Wq2_r, Wq3_r, bqr_r, wa_r,
                up_r):
    lr = lambda x: jnp.where(x > 0, x, 0.01 * x)
    bf = lambda x: x.astype(jnp.bfloat16)
    dot = lambda a, b: jnp.dot(bf(a), b, preferred_element_type=jnp.float32)
    rel = rel_r[...]
    hs = hs_r[...]
    hqr = hqr_r[...]
    head = head_r[...]
    tail = tail_r[...]
    h1 = lr(dot(rel, W1a_r[...]) + tpw_r[...])
    h2 = lr(dot(h1, W2_r[...]) + b2_r[...])
    hr = h2 + rel
    gin = (dot(hr, Wg1_r[...])
           + 0.25 * (dot(hqr, Wg2_r[...]) + dot(head, Wg3_r[...])
                     + dot(tail, Wg4_r[...]))
           + dot(hs, Wg5_r[...]) + bg_r[...])
    gates = jax.nn.sigmoid(gin)
    update = gates[:, :D]
    reset = gates[:, D:]
    cand = jnp.tanh(dot(hr, Wt1_r[...]) + dot(reset * hs, Wt2_r[...]) + bt_r[...])
    message = (1.0 - update) * hs + update * cand
    att = lr(dot(hs, Ws_r[...]) + dot(hr, Wr_r[...]) + dot(hqr, Wq1_r[...])
             + dot(head, Wq2_r[...]) + dot(tail, Wq3_r[...]) + bqr_r[...])
    alpha = dot(att, wa_r[...])
    up_r[...] = jax.nn.sigmoid(alpha) * message


def _dense_stage(rel, tpw_g, hs, hqr, head, tail, weights):
    nblk = E // BLK
    row_spec = lambda w: pl.BlockSpec((BLK, w), lambda i: (i, 0))
    full = lambda a: pl.BlockSpec(a.shape, lambda i: (0,) * a.ndim)
    return pl.pallas_call(
        _dense_body,
        grid=(nblk,),
        in_specs=[row_spec(D), row_spec(D), row_spec(D), row_spec(D),
                  row_spec(D), row_spec(D)] + [full(w) for w in weights],
        out_specs=row_spec(D),
        out_shape=jax.ShapeDtypeStruct((E, D), jnp.float32),
    )(rel, tpw_g, hs, hqr, head, tail, *weights)


def _scatter_body(obj_h, up_h, ones_h, zu_h,
                  pu_o, pd_o,
                  obj_v, up_v, acc):
    cid = lax.axis_index("c")
    sid = lax.axis_index("s")
    r0 = sid * NPT
    pltpu.sync_copy(zu_h.at[pl.ds(r0, NPT)], acc.at[pl.ds(r0, NPT)])
    plsc.subcore_barrier()

    @pl.when(cid == 0)
    def _up_core():
        def body(i, carry):
            base = (sid + i * NS) * GB
            pltpu.sync_copy(obj_h.at[pl.ds(base, GB)], obj_v)
            pltpu.sync_copy(up_h.at[pl.ds(base, GB)], up_v)
            pltpu.sync_copy(up_v, acc.at[obj_v], add=True)
            return carry

        lax.fori_loop(0, _share(NBLK, NS, sid), body, 0)

    @pl.when(cid == 1)
    def _deg_core():
        pltpu.sync_copy(ones_h, up_v)

        def body(i, carry):
            base = (sid + i * NS) * GB
            pltpu.sync_copy(obj_h.at[pl.ds(base, GB)], obj_v)
            pltpu.sync_copy(up_v, acc.at[obj_v], add=True)
            return carry

        lax.fori_loop(0, _share(NBLK, NS, sid), body, 0)

    plsc.subcore_barrier()

    @pl.when(cid == 0)
    def _out_up():
        pltpu.sync_copy(acc.at[pl.ds(r0, NPT)], pu_o.at[pl.ds(r0, NPT)])

    @pl.when(cid == 1)
    def _out_deg():
        pltpu.sync_copy(acc.at[pl.ds(r0, NPT)], pd_o.at[pl.ds(r0, NPT)])


def _scatter_stage(obj, up):
    f32 = jnp.float32
    ones = jnp.ones((GB, D), f32)
    zu = jnp.zeros((NP, D), f32)
    return pl.kernel(
        _scatter_body,
        out_type=[jax.ShapeDtypeStruct((NP, D), f32),
                  jax.ShapeDtypeStruct((NP, D), f32)],
        mesh=_sc_mesh(),
        scratch_types=[pltpu.VMEM((GB,), jnp.int32),
                       pltpu.VMEM((GB, D), f32),
                       pltpu.VMEM_SHARED((NP, D), f32)],
    )(obj, up, ones, zu)


def _final_body(pu_r, pd_r, Wh_r, out_r):
    deg = pd_r[:, 0:1]
    agg = pu_r[...] / jnp.sqrt(deg + 0.0001)
    out_r[...] = jnp.dot(agg, Wh_r[...], preferred_element_type=jnp.float32)


def _final_stage(part_up, part_deg, Wh):
    rb = 2000
    return pl.pallas_call(
        _final_body,
        grid=(N // rb,),
        in_specs=[pl.BlockSpec((rb, D), lambda i: (i, 0)),
                  pl.BlockSpec((rb, D), lambda i: (i, 0)),
                  pl.BlockSpec(Wh.shape, lambda i: (0, 0))],
        out_specs=pl.BlockSpec((rb, D), lambda i: (i, 0)),
        out_shape=jax.ShapeDtypeStruct((N, D), jnp.float32),
    )(part_up, part_deg, Wh)


def kernel(q_sub, q_rel, hidden, edges, n_node, edge_head_rc_repr,
           edge_tail_rc_repr, query_head_rc_repr, rela_embed, time_pe,
           Ws, Wr, W1f, b1f, W2f, b2f, Wqr, bqr, w_alpha, Wg, bg,
           Wt, bt, Wh):
    sub = edges[:, 4]
    obj = edges[:, 5]
    r_idx = edges[:, 0]
    e2 = edges[:, 2]
    e6 = edges[:, 6]
    obj = obj + (jnp.asarray(n_node, dtype=obj.dtype) - N)

    q_rel_pad = jnp.pad(q_rel.astype(jnp.int32), (0, NQP - q_rel.shape[0]))
    qr_table = _qr_stage(q_rel_pad, rela_embed)

    tpw = _tpw_stage(time_pe, W1f[D:], b1f.reshape(1, D))
    hs, rel, tpw_g, hqr = _gather_stage(sub, e2, e6, r_idx, hidden,
                                        rela_embed, tpw, qr_table)

    b16 = lambda a: a.astype(jnp.bfloat16)
    weights = (
        b16(W1f[:D]), b16(W2f), b2f.reshape(1, D),
        b16(Wg[:D]), b16(Wg[D:2 * D]), b16(Wg[2 * D:3 * D]),
        b16(Wg[3 * D:4 * D]), b16(Wg[4 * D:]), bg.reshape(1, 2 * D),
        b16(Wt[:D]), b16(Wt[D:]), bt.reshape(1, D),
        b16(Ws), b16(Wr), b16(Wqr[:D]), b16(Wqr[D:2 * D]), b16(Wqr[2 * D:]),
        bqr.reshape(1, D), b16(w_alpha),
    )
    up = _dense_stage(rel, tpw_g, hs, hqr,
                      edge_head_rc_repr, edge_tail_rc_repr, weights)

    part_up, part_deg = _scatter_stage(obj.astype(jnp.int32), up)

    return _final_stage(part_up, part_deg, Wh)


# speedup vs baseline: 1.3907x; 1.2080x over previous
import functools

import jax
import jax.numpy as jnp
from jax import lax
from jax.experimental import pallas as pl
from jax.experimental.pallas import tpu as pltpu
from jax.experimental.pallas import tpu_sc as plsc

E = 160000
D = 128
TD = 32
N = 10000
NQP = 10240
BLK = 2048
GB = 128
NC, NS = 2, 16
NW = NC * NS
NP = 10240
NPT = NP // NS
CHUNK = 40960


@functools.lru_cache(maxsize=1)
def _sc_mesh():
    return plsc.VectorSubcoreMesh(
        core_axis_name="c", subcore_axis_name="s",
        num_cores=NC, num_subcores=NS)


def _share(total, workers, w):
    return total // workers + jnp.where(w < total % workers, 1, 0)


def _qr_body(qrel_h, rela_h, qrt_o, idx_v, rows_v):
    cid = lax.axis_index("c")
    sid = lax.axis_index("s")
    wid = sid * NC + cid

    def body(i, carry):
        base = (wid + i * NW) * GB
        pltpu.sync_copy(qrel_h.at[pl.ds(base, GB)], idx_v)
        pltpu.sync_copy(rela_h.at[idx_v], rows_v)
        pltpu.sync_copy(rows_v, qrt_o.at[pl.ds(base, GB)])
        return carry

    lax.fori_loop(0, _share(NQP // GB, NW, wid), body, 0)


def _qr_stage(q_rel_pad, rela_embed):
    return pl.kernel(
        _qr_body,
        out_type=jax.ShapeDtypeStruct((NQP, D), jnp.float32),
        mesh=_sc_mesh(),
        scratch_types=[pltpu.VMEM((GB,), jnp.int32),
                       pltpu.VMEM((GB, D), jnp.float32)],
    )(q_rel_pad, rela_embed)


def _tpw_body(tp_r, W1b_r, b1_r, out_r):
    out_r[...] = (jnp.dot(tp_r[...], W1b_r[...],
                          preferred_element_type=jnp.float32) + b1_r[...])


def _tpw_stage(time_pe, W1b, b1f):
    rb = 2000
    return pl.pallas_call(
        _tpw_body,
        grid=(N // rb,),
        in_specs=[pl.BlockSpec((rb, TD), lambda i: (i, 0)),
                  pl.BlockSpec((TD, D), lambda i: (0, 0)),
                  pl.BlockSpec((1, D), lambda i: (0, 0))],
        out_specs=pl.BlockSpec((rb, D), lambda i: (i, 0)),
        out_shape=jax.ShapeDtypeStruct((N, D), jnp.float32),
    )(time_pe, W1b, b1f)


def _gather_body(nblk_total, sub_h, e2_h, e6_h, ridx_h,
                 hidden_h, rela_h, tpw_h, qrt_h,
                 hs_o, rel_o, tpw_o, hqr_o,
                 sub_v, e2_v, e6_v, ridx_v,
                 hs_v, rel_v, tpw_v, hqr_v, s0, s1, s2, s3):
    cid = lax.axis_index("c")
    sid = lax.axis_index("s")
    wid = sid * NC + cid

    def body(i, carry):
        base = (wid + i * NW) * GB
        pltpu.sync_copy(sub_h.at[pl.ds(base, GB)], sub_v)
        pltpu.sync_copy(e2_h.at[pl.ds(base, GB)], e2_v)
        pltpu.sync_copy(e6_h.at[pl.ds(base, GB)], e6_v)
        pltpu.sync_copy(ridx_h.at[pl.ds(base, GB)], ridx_v)
        c0 = pltpu.async_copy(hidden_h.at[sub_v], hs_v, s0)
        c1 = pltpu.async_copy(rela_h.at[e2_v], rel_v, s1)
        c2 = pltpu.async_copy(tpw_h.at[e6_v], tpw_v, s2)
        c3 = pltpu.async_copy(qrt_h.at[ridx_v], hqr_v, s3)
        c0.wait()
        pltpu.sync_copy(hs_v, hs_o.at[pl.ds(base, GB)])
        c1.wait()
        pltpu.sync_copy(rel_v, rel_o.at[pl.ds(base, GB)])
        c2.wait()
        pltpu.sync_copy(tpw_v, tpw_o.at[pl.ds(base, GB)])
        c3.wait()
        pltpu.sync_copy(hqr_v, hqr_o.at[pl.ds(base, GB)])
        return carry

    lax.fori_loop(0, _share(nblk_total, NW, wid), body, 0)


def _gather_stage(sub, e2, e6, r_idx, hidden, rela_embed, tpw, qr_table):
    f32 = jnp.float32
    i32 = jnp.int32
    ce = sub.shape[0]
    return pl.kernel(
        functools.partial(_gather_body, ce // GB),
        out_type=[jax.ShapeDtypeStruct((ce, D), f32),
                  jax.ShapeDtypeStruct((ce, D), f32),
                  jax.ShapeDtypeStruct((ce, D), f32),
                  jax.ShapeDtypeStruct((ce, D), f32)],
        mesh=_sc_mesh(),
        scratch_types=[pltpu.VMEM((GB,), i32), pltpu.VMEM((GB,), i32),
                       pltpu.VMEM((GB,), i32), pltpu.VMEM((GB,), i32),
                       pltpu.VMEM((GB, D), f32), pltpu.VMEM((GB, D), f32),
                       pltpu.VMEM((GB, D), f32), pltpu.VMEM((GB, D), f32),
                       pltpu.SemaphoreType.DMA, pltpu.SemaphoreType.DMA,
                       pltpu.SemaphoreType.DMA, pltpu.SemaphoreType.DMA],
    )(sub, e2, e6, r_idx, hidden, rela_embed, tpw, qr_table)


def _dense_body(rel_r, tpw_r, hs_r, hqr_r, head_r, tail_r,
                W1a_r, W2_r, b2_r,
                Wg1_r, Wg2_r, Wg3_r, Wg4_r, Wg5_r, bg_r,
                Wt1_r, Wt2_r, bt_r,
                Ws_r, Wr_r, Wq1_r, Wq2_r, Wq3_r, bqr_r, wa_r,
                up_r):
    lr = lambda x: jnp.where(x > 0, x, 0.01 * x)
    dot = lambda a, b: jnp.dot(a, b, preferred_element_type=jnp.float32)
    rel = rel_r[...]
    hs = hs_r[...]
    hqr = hqr_r[...]
    head = head_r[...]
    tail = tail_r[...]
    h1 = lr(dot(rel, W1a_r[...]) + tpw_r[...])
    h2 = lr(dot(h1, W2_r[...]) + b2_r[...])
    hr = h2 + rel
    gin = (dot(hr, Wg1_r[...])
           + 0.25 * (dot(hqr, Wg2_r[...]) + dot(head, Wg3_r[...])
                     + dot(tail, Wg4_r[...]))
           + dot(hs, Wg5_r[...]) + bg_r[...])
    gates = jax.nn.sigmoid(gin)
    update = gates[:, :D]
    reset = gates[:, D:]
    cand = jnp.tanh(dot(hr, Wt1_r[...]) + dot(reset * hs, Wt2_r[...]) + bt_r[...])
    message = (1.0 - update) * hs + update * cand
    att = lr(dot(hs, Ws_r[...]) + dot(hr, Wr_r[...]) + dot(hqr, Wq1_r[...])
             + dot(head, Wq2_r[...]) + dot(tail, Wq3_r[...]) + bqr_r[...])
    alpha = dot(att, wa_r[...])
    up_r[...] = jax.nn.sigmoid(alpha) * message


def _dense_stage(rel, tpw_g, hs, hqr, head, tail, weights):
    ce = rel.shape[0]
    nblk = (ce + BLK - 1) // BLK
    row_spec = lambda w: pl.BlockSpec((BLK, w), lambda i: (i, 0))
    full = lambda a: pl.BlockSpec(a.shape, lambda i: (0,) * a.ndim)
    return pl.pallas_call(
        _dense_body,
        grid=(nblk,),
        in_specs=[row_spec(D), row_spec(D), row_spec(D), row_spec(D),
                  row_spec(D), row_spec(D)] + [full(w) for w in weights],
        out_specs=row_spec(D),
        out_shape=jax.ShapeDtypeStruct((ce, D), jnp.float32),
    )(rel, tpw_g, hs, hqr, head, tail, *weights)


def _scatter_body(nblk_total, obj_h, up_h, ones_h, ipu_h, ipd_h,
                  pu_o, pd_o,
                  obj_v, up_v, acc):
    cid = lax.axis_index("c")
    sid = lax.axis_index("s")
    r0 = sid * NPT

    @pl.when(cid == 0)
    def _init_up():
        pltpu.sync_copy(ipu_h.at[pl.ds(r0, NPT)], acc.at[pl.ds(r0, NPT)])

    @pl.when(cid == 1)
    def _init_deg():
        pltpu.sync_copy(ipd_h.at[pl.ds(r0, NPT)], acc.at[pl.ds(r0, NPT)])

    plsc.subcore_barrier()

    @pl.when(cid == 0)
    def _up_core():
        def body(i, carry):
            base = (sid + i * NS) * GB
            pltpu.sync_copy(obj_h.at[pl.ds(base, GB)], obj_v)
            pltpu.sync_copy(up_h.at[pl.ds(base, GB)], up_v)
            pltpu.sync_copy(up_v, acc.at[obj_v], add=True)
            return carry

        lax.fori_loop(0, _share(nblk_total, NS, sid), body, 0)

    @pl.when(cid == 1)
    def _deg_core():
        pltpu.sync_copy(ones_h, up_v)

        def body(i, carry):
            base = (sid + i * NS) * GB
            pltpu.sync_copy(obj_h.at[pl.ds(base, GB)], obj_v)
            pltpu.sync_copy(up_v, acc.at[obj_v], add=True)
            return carry

        lax.fori_loop(0, _share(nblk_total, NS, sid), body, 0)

    plsc.subcore_barrier()

    @pl.when(cid == 0)
    def _out_up():
        pltpu.sync_copy(acc.at[pl.ds(r0, NPT)], pu_o.at[pl.ds(r0, NPT)])

    @pl.when(cid == 1)
    def _out_deg():
        pltpu.sync_copy(acc.at[pl.ds(r0, NPT)], pd_o.at[pl.ds(r0, NPT)])


def _scatter_stage(obj, up, init_pu, init_pd):
    f32 = jnp.float32
    ones = jnp.ones((GB, D), f32)
    ce = obj.shape[0]
    return pl.kernel(
        functools.partial(_scatter_body, ce // GB),
        out_type=[jax.ShapeDtypeStruct((NP, D), f32),
                  jax.ShapeDtypeStruct((NP, D), f32)],
        mesh=_sc_mesh(),
        scratch_types=[pltpu.VMEM((GB,), jnp.int32),
                       pltpu.VMEM((GB, D), f32),
                       pltpu.VMEM_SHARED((NP, D), f32)],
    )(obj, up, ones, init_pu, init_pd)


def _final_body(pu_r, pd_r, Wh_r, out_r):
    deg = pd_r[:, 0:1]
    agg = pu_r[...] / jnp.sqrt(deg + 0.0001)
    out_r[...] = jnp.dot(agg, Wh_r[...], preferred_element_type=jnp.float32)


def _final_stage(part_up, part_deg, Wh):
    rb = 2000
    return pl.pallas_call(
        _final_body,
        grid=(N // rb,),
        in_specs=[pl.BlockSpec((rb, D), lambda i: (i, 0)),
                  pl.BlockSpec((rb, D), lambda i: (i, 0)),
                  pl.BlockSpec(Wh.shape, lambda i: (0, 0))],
        out_specs=pl.BlockSpec((rb, D), lambda i: (i, 0)),
        out_shape=jax.ShapeDtypeStruct((N, D), jnp.float32),
    )(part_up, part_deg, Wh)


def kernel(q_sub, q_rel, hidden, edges, n_node, edge_head_rc_repr,
           edge_tail_rc_repr, query_head_rc_repr, rela_embed, time_pe,
           Ws, Wr, W1f, b1f, W2f, b2f, Wqr, bqr, w_alpha, Wg, bg,
           Wt, bt, Wh):
    f32 = jnp.float32
    sub = edges[:, 4].astype(jnp.int32)
    obj = edges[:, 5].astype(jnp.int32)
    r_idx = edges[:, 0].astype(jnp.int32)
    e2 = edges[:, 2].astype(jnp.int32)
    e6 = edges[:, 6].astype(jnp.int32)
    obj = obj + (jnp.asarray(n_node, dtype=obj.dtype) - N)

    q_rel_pad = jnp.pad(q_rel.astype(jnp.int32), (0, NQP - q_rel.shape[0]))
    qr_table = _qr_stage(q_rel_pad, rela_embed)
    tpw = _tpw_stage(time_pe, W1f[D:], b1f.reshape(1, D))

    weights = (
        W1f[:D], W2f, b2f.reshape(1, D),
        Wg[:D], Wg[D:2 * D], Wg[2 * D:3 * D], Wg[3 * D:4 * D], Wg[4 * D:],
        bg.reshape(1, 2 * D),
        Wt[:D], Wt[D:], bt.reshape(1, D),
        Ws, Wr, Wqr[:D], Wqr[D:2 * D], Wqr[2 * D:], bqr.reshape(1, D),
        w_alpha,
    )

    pu = jnp.zeros((NP, D), f32)
    pd = jnp.zeros((NP, D), f32)
    for lo in range(0, E, CHUNK):
        hi = min(lo + CHUNK, E)
        hs_c, rel_c, tpw_c, hqr_c = _gather_stage(
            sub[lo:hi], e2[lo:hi], e6[lo:hi], r_idx[lo:hi],
            hidden, rela_embed, tpw, qr_table)
        up_c = _dense_stage(rel_c, tpw_c, hs_c, hqr_c,
                            edge_head_rc_repr[lo:hi],
                            edge_tail_rc_repr[lo:hi], weights)
        pu, pd = _scatter_stage(obj[lo:hi], up_c, pu, pd)

    return _final_stage(pu, pd, Wh)


# speedup vs baseline: 1.4322x; 1.0298x over previous
import functools

import jax
import jax.numpy as jnp
from jax import lax
from jax.experimental import pallas as pl
from jax.experimental.pallas import tpu as pltpu
from jax.experimental.pallas import tpu_sc as plsc

E = 160000
D = 128
TD = 32
N = 10000
NQP = 10240
BLK = 2048
GB = 128
NC, NS = 2, 16
NW = NC * NS
NP = 10240
NPT = NP // NS
CHUNK = 40960


@functools.lru_cache(maxsize=1)
def _sc_mesh():
    return plsc.VectorSubcoreMesh(
        core_axis_name="c", subcore_axis_name="s",
        num_cores=NC, num_subcores=NS)


def _share(total, workers, w):
    return total // workers + jnp.where(w < total % workers, 1, 0)


def _qr_body(qrel_h, rela_h, qrt_o, idx_v, rows_v):
    cid = lax.axis_index("c")
    sid = lax.axis_index("s")
    wid = sid * NC + cid

    def body(i, carry):
        base = (wid + i * NW) * GB
        pltpu.sync_copy(qrel_h.at[pl.ds(base, GB)], idx_v)
        pltpu.sync_copy(rela_h.at[idx_v], rows_v)
        pltpu.sync_copy(rows_v, qrt_o.at[pl.ds(base, GB)])
        return carry

    lax.fori_loop(0, _share(NQP // GB, NW, wid), body, 0)


def _qr_stage(q_rel_pad, rela_embed):
    return pl.kernel(
        _qr_body,
        out_type=jax.ShapeDtypeStruct((NQP, D), jnp.float32),
        mesh=_sc_mesh(),
        scratch_types=[pltpu.VMEM((GB,), jnp.int32),
                       pltpu.VMEM((GB, D), jnp.float32)],
    )(q_rel_pad, rela_embed)


def _tpw_body(tp_r, W1b_r, b1_r, out_r):
    out_r[...] = (jnp.dot(tp_r[...], W1b_r[...],
                          preferred_element_type=jnp.float32) + b1_r[...])


def _tpw_stage(time_pe, W1b, b1f):
    rb = 2000
    return pl.pallas_call(
        _tpw_body,
        grid=(N // rb,),
        in_specs=[pl.BlockSpec((rb, TD), lambda i: (i, 0)),
                  pl.BlockSpec((TD, D), lambda i: (0, 0)),
                  pl.BlockSpec((1, D), lambda i: (0, 0))],
        out_specs=pl.BlockSpec((rb, D), lambda i: (i, 0)),
        out_shape=jax.ShapeDtypeStruct((N, D), jnp.float32),
    )(time_pe, W1b, b1f)


def _gather_body(nblk_total, idx_h, hidden_h, rela_h, tpw_h, qrt_h,
                 hs_o, rel_o, tpw_o, hqr_o,
                 idx_v, hs_v, rel_v, tpw_v, hqr_v, s0, s1, s2, s3):
    cid = lax.axis_index("c")
    sid = lax.axis_index("s")
    wid = sid * NC + cid

    def body(i, carry):
        b = wid + i * NW
        base = b * GB
        pltpu.sync_copy(idx_h.at[b], idx_v)
        c0 = pltpu.async_copy(hidden_h.at[idx_v.at[0]], hs_v, s0)
        c1 = pltpu.async_copy(rela_h.at[idx_v.at[1]], rel_v, s1)
        c2 = pltpu.async_copy(tpw_h.at[idx_v.at[2]], tpw_v, s2)
        c3 = pltpu.async_copy(qrt_h.at[idx_v.at[3]], hqr_v, s3)
        c0.wait()
        pltpu.sync_copy(hs_v, hs_o.at[pl.ds(base, GB)])
        c1.wait()
        pltpu.sync_copy(rel_v, rel_o.at[pl.ds(base, GB)])
        c2.wait()
        pltpu.sync_copy(tpw_v, tpw_o.at[pl.ds(base, GB)])
        c3.wait()
        pltpu.sync_copy(hqr_v, hqr_o.at[pl.ds(base, GB)])
        return carry

    lax.fori_loop(0, _share(nblk_total, NW, wid), body, 0)


def _gather_stage(idx_all, hidden, rela_embed, tpw, qr_table):
    f32 = jnp.float32
    i32 = jnp.int32
    nb = idx_all.shape[0]
    ce = nb * GB
    return pl.kernel(
        functools.partial(_gather_body, nb),
        out_type=[jax.ShapeDtypeStruct((ce, D), f32),
                  jax.ShapeDtypeStruct((ce, D), f32),
                  jax.ShapeDtypeStruct((ce, D), f32),
                  jax.ShapeDtypeStruct((ce, D), f32)],
        mesh=_sc_mesh(),
        scratch_types=[pltpu.VMEM((4, GB), i32),
                       pltpu.VMEM((GB, D), f32), pltpu.VMEM((GB, D), f32),
                       pltpu.VMEM((GB, D), f32), pltpu.VMEM((GB, D), f32),
                       pltpu.SemaphoreType.DMA, pltpu.SemaphoreType.DMA,
                       pltpu.SemaphoreType.DMA, pltpu.SemaphoreType.DMA],
    )(idx_all, hidden, rela_embed, tpw, qr_table)


def _dense_body(rel_r, tpw_r, hs_r, hqr_r, head_r, tail_r,
                W1a_r, W2_r, b2_r,
                Wg1_r, Wg2_r, Wg3_r, Wg4_r, Wg5_r, bg_r,
                Wt1_r, Wt2_r, bt_r,
                Ws_r, Wr_r, Wq1_r, Wq2_r, Wq3_r, bqr_r, wa_r,
                up_r):
    lr = lambda x: jnp.where(x > 0, x, 0.01 * x)
    dot = lambda a, b: jnp.dot(a, b, preferred_element_type=jnp.float32)
    rel = rel_r[...]
    hs = hs_r[...]
    hqr = hqr_r[...]
    head = head_r[...]
    tail = tail_r[...]
    h1 = lr(dot(rel, W1a_r[...]) + tpw_r[...])
    h2 = lr(dot(h1, W2_r[...]) + b2_r[...])
    hr = h2 + rel
    gin = (dot(hr, Wg1_r[...])
           + 0.25 * (dot(hqr, Wg2_r[...]) + dot(head, Wg3_r[...])
                     + dot(tail, Wg4_r[...]))
           + dot(hs, Wg5_r[...]) + bg_r[...])
    gates = jax.nn.sigmoid(gin)
    update = gates[:, :D]
    reset = gates[:, D:]
    cand = jnp.tanh(dot(hr, Wt1_r[...]) + dot(reset * hs, Wt2_r[...]) + bt_r[...])
    message = (1.0 - update) * hs + update * cand
    att = lr(dot(hs, Ws_r[...]) + dot(hr, Wr_r[...]) + dot(hqr, Wq1_r[...])
             + dot(head, Wq2_r[...]) + dot(tail, Wq3_r[...]) + bqr_r[...])
    alpha = dot(att, wa_r[...])
    up_r[...] = jax.nn.sigmoid(alpha) * message


def _dense_stage(rel, tpw_g, hs, hqr, head, tail, weights):
    ce = rel.shape[0]
    nblk = (ce + BLK - 1) // BLK
    row_spec = lambda w: pl.BlockSpec((BLK, w), lambda i: (i, 0))
    full = lambda a: pl.BlockSpec(a.shape, lambda i: (0,) * a.ndim)
    return pl.pallas_call(
        _dense_body,
        grid=(nblk,),
        in_specs=[row_spec(D), row_spec(D), row_spec(D), row_spec(D),
                  row_spec(D), row_spec(D)] + [full(w) for w in weights],
        out_specs=row_spec(D),
        out_shape=jax.ShapeDtypeStruct((ce, D), jnp.float32),
    )(rel, tpw_g, hs, hqr, head, tail, *weights)


def _scatter_body(nblk_total, obj_h, up_h, ones_h, ipu_h, ipd_h,
                  pu_o, pd_o,
                  obj_v, up_v, acc):
    cid = lax.axis_index("c")
    sid = lax.axis_index("s")
    r0 = sid * NPT

    @pl.when(cid == 0)
    def _init_up():
        pltpu.sync_copy(ipu_h.at[pl.ds(r0, NPT)], acc.at[pl.ds(r0, NPT)])

    @pl.when(cid == 1)
    def _init_deg():
        pltpu.sync_copy(ipd_h.at[pl.ds(r0, NPT)], acc.at[pl.ds(r0, NPT)])

    plsc.subcore_barrier()

    @pl.when(cid == 0)
    def _up_core():
        def body(i, carry):
            base = (sid + i * NS) * GB
            pltpu.sync_copy(obj_h.at[pl.ds(base, GB)], obj_v)
            pltpu.sync_copy(up_h.at[pl.ds(base, GB)], up_v)
            pltpu.sync_copy(up_v, acc.at[obj_v], add=True)
            return carry

        lax.fori_loop(0, _share(nblk_total, NS, sid), body, 0)

    @pl.when(cid == 1)
    def _deg_core():
        pltpu.sync_copy(ones_h, up_v)

        def body(i, carry):
            base = (sid + i * NS) * GB
            pltpu.sync_copy(obj_h.at[pl.ds(base, GB)], obj_v)
            pltpu.sync_copy(up_v, acc.at[obj_v], add=True)
            return carry

        lax.fori_loop(0, _share(nblk_total, NS, sid), body, 0)

    plsc.subcore_barrier()

    @pl.when(cid == 0)
    def _out_up():
        pltpu.sync_copy(acc.at[pl.ds(r0, NPT)], pu_o.at[pl.ds(r0, NPT)])

    @pl.when(cid == 1)
    def _out_deg():
        pltpu.sync_copy(acc.at[pl.ds(r0, NPT)], pd_o.at[pl.ds(r0, NPT)])


def _scatter_stage(obj, up, init_pu, init_pd):
    f32 = jnp.float32
    ones = jnp.ones((GB, D), f32)
    ce = obj.shape[0]
    return pl.kernel(
        functools.partial(_scatter_body, ce // GB),
        out_type=[jax.ShapeDtypeStruct((NP, D), f32),
                  jax.ShapeDtypeStruct((NP, D), f32)],
        mesh=_sc_mesh(),
        scratch_types=[pltpu.VMEM((GB,), jnp.int32),
                       pltpu.VMEM((GB, D), f32),
                       pltpu.VMEM_SHARED((NP, D), f32)],
    )(obj, up, ones, init_pu, init_pd)


def _final_body(pu_r, pd_r, Wh_r, out_r):
    deg = pd_r[:, 0:1]
    agg = pu_r[...] / jnp.sqrt(deg + 0.0001)
    out_r[...] = jnp.dot(agg, Wh_r[...], preferred_element_type=jnp.float32)


def _final_stage(part_up, part_deg, Wh):
    rb = 2000
    return pl.pallas_call(
        _final_body,
        grid=(N // rb,),
        in_specs=[pl.BlockSpec((rb, D), lambda i: (i, 0)),
                  pl.BlockSpec((rb, D), lambda i: (i, 0)),
                  pl.BlockSpec(Wh.shape, lambda i: (0, 0))],
        out_specs=pl.BlockSpec((rb, D), lambda i: (i, 0)),
        out_shape=jax.ShapeDtypeStruct((N, D), jnp.float32),
    )(part_up, part_deg, Wh)


def kernel(q_sub, q_rel, hidden, edges, n_node, edge_head_rc_repr,
           edge_tail_rc_repr, query_head_rc_repr, rela_embed, time_pe,
           Ws, Wr, W1f, b1f, W2f, b2f, Wqr, bqr, w_alpha, Wg, bg,
           Wt, bt, Wh):
    f32 = jnp.float32
    sub = edges[:, 4].astype(jnp.int32)
    obj = edges[:, 5].astype(jnp.int32)
    r_idx = edges[:, 0].astype(jnp.int32)
    e2 = edges[:, 2].astype(jnp.int32)
    e6 = edges[:, 6].astype(jnp.int32)
    obj = obj + (jnp.asarray(n_node, dtype=obj.dtype) - N)

    q_rel_pad = jnp.pad(q_rel.astype(jnp.int32), (0, NQP - q_rel.shape[0]))
    qr_table = _qr_stage(q_rel_pad, rela_embed)
    tpw = _tpw_stage(time_pe, W1f[D:], b1f.reshape(1, D))

    weights = (
        W1f[:D], W2f, b2f.reshape(1, D),
        Wg[:D], Wg[D:2 * D], Wg[2 * D:3 * D], Wg[3 * D:4 * D], Wg[4 * D:],
        bg.reshape(1, 2 * D),
        Wt[:D], Wt[D:], bt.reshape(1, D),
        Ws, Wr, Wqr[:D], Wqr[D:2 * D], Wqr[2 * D:], bqr.reshape(1, D),
        w_alpha,
    )

    pu = jnp.zeros((NP, D), f32)
    pd = jnp.zeros((NP, D), f32)
    for lo in range(0, E, CHUNK):
        hi = min(lo + CHUNK, E)
        nb = (hi - lo) // GB
        idx_all = jnp.stack([sub[lo:hi].reshape(nb, GB),
                             e2[lo:hi].reshape(nb, GB),
                             e6[lo:hi].reshape(nb, GB),
                             r_idx[lo:hi].reshape(nb, GB)], axis=1)
        hs_c, rel_c, tpw_c, hqr_c = _gather_stage(
            idx_all, hidden, rela_embed, tpw, qr_table)
        up_c = _dense_stage(rel_c, tpw_c, hs_c, hqr_c,
                            edge_head_rc_repr[lo:hi],
                            edge_tail_rc_repr[lo:hi], weights)
        pu, pd = _scatter_stage(obj[lo:hi], up_c, pu, pd)

    return _final_stage(pu, pd, Wh)


# speedup vs baseline: 1.4750x; 1.0299x over previous
import functools

import jax
import jax.numpy as jnp
from jax import lax
from jax.experimental import pallas as pl
from jax.experimental.pallas import tpu as pltpu
from jax.experimental.pallas import tpu_sc as plsc

E = 160000
D = 128
TD = 32
N = 10000
NQP = 10240
BLK = 2048
GB = 128
NC, NS = 2, 16
NW = NC * NS
NP = 10240
NPT = NP // NS
CHUNK = 40960


@functools.lru_cache(maxsize=1)
def _sc_mesh():
    return plsc.VectorSubcoreMesh(
        core_axis_name="c", subcore_axis_name="s",
        num_cores=NC, num_subcores=NS)


def _share(total, workers, w):
    return total // workers + jnp.where(w < total % workers, 1, 0)


def _qr_body(qrel_h, rela_h, qrt_o, idx_v, rows_v):
    cid = lax.axis_index("c")
    sid = lax.axis_index("s")
    wid = sid * NC + cid

    def body(i, carry):
        base = (wid + i * NW) * GB
        pltpu.sync_copy(qrel_h.at[pl.ds(base, GB)], idx_v)
        pltpu.sync_copy(rela_h.at[idx_v], rows_v)
        pltpu.sync_copy(rows_v, qrt_o.at[pl.ds(base, GB)])
        return carry

    lax.fori_loop(0, _share(NQP // GB, NW, wid), body, 0)


def _qr_stage(q_rel_pad, rela_embed):
    return pl.kernel(
        _qr_body,
        out_type=jax.ShapeDtypeStruct((NQP, D), jnp.float32),
        mesh=_sc_mesh(),
        scratch_types=[pltpu.VMEM((GB,), jnp.int32),
                       pltpu.VMEM((GB, D), jnp.float32)],
    )(q_rel_pad, rela_embed)


def _tpw_body(tp_r, W1b_r, b1_r, out_r):
    out_r[...] = (jnp.dot(tp_r[...], W1b_r[...],
                          preferred_element_type=jnp.float32) + b1_r[...])


def _tpw_stage(time_pe, W1b, b1f):
    rb = 2000
    return pl.pallas_call(
        _tpw_body,
        grid=(N // rb,),
        in_specs=[pl.BlockSpec((rb, TD), lambda i: (i, 0)),
                  pl.BlockSpec((TD, D), lambda i: (0, 0)),
                  pl.BlockSpec((1, D), lambda i: (0, 0))],
        out_specs=pl.BlockSpec((rb, D), lambda i: (i, 0)),
        out_shape=jax.ShapeDtypeStruct((N, D), jnp.float32),
    )(time_pe, W1b, b1f)


def _gather_body(nblk_total, idx_h, hidden_h, rela_h, tpw_h, qrt_h,
                 hs_o, rel_o, tpw_o, hqr_o,
                 idx_v, hs_v, rel_v, tpw_v, hqr_v,
                 s0, s1, s2, s3, t0, t1, t2, t3):
    cid = lax.axis_index("c")
    sid = lax.axis_index("s")
    wid = sid * NC + cid

    def body(i, carry):
        b = wid + i * NW
        base = b * GB
        pltpu.sync_copy(idx_h.at[b], idx_v)
        c0 = pltpu.async_copy(hidden_h.at[idx_v.at[0]], hs_v, s0)
        c1 = pltpu.async_copy(rela_h.at[idx_v.at[1]], rel_v, s1)
        c2 = pltpu.async_copy(tpw_h.at[idx_v.at[2]], tpw_v, s2)
        c3 = pltpu.async_copy(qrt_h.at[idx_v.at[3]], hqr_v, s3)
        c0.wait()
        w0 = pltpu.async_copy(hs_v, hs_o.at[pl.ds(base, GB)], t0)
        c1.wait()
        w1 = pltpu.async_copy(rel_v, rel_o.at[pl.ds(base, GB)], t1)
        c2.wait()
        w2 = pltpu.async_copy(tpw_v, tpw_o.at[pl.ds(base, GB)], t2)
        c3.wait()
        w3 = pltpu.async_copy(hqr_v, hqr_o.at[pl.ds(base, GB)], t3)
        w0.wait()
        w1.wait()
        w2.wait()
        w3.wait()
        return carry

    lax.fori_loop(0, _share(nblk_total, NW, wid), body, 0)


def _gather_stage(idx_all, hidden, rela_embed, tpw, qr_table):
    f32 = jnp.float32
    i32 = jnp.int32
    nb = idx_all.shape[0]
    ce = nb * GB
    return pl.kernel(
        functools.partial(_gather_body, nb),
        out_type=[jax.ShapeDtypeStruct((ce, D), f32),
                  jax.ShapeDtypeStruct((ce, D), f32),
                  jax.ShapeDtypeStruct((ce, D), f32),
                  jax.ShapeDtypeStruct((ce, D), f32)],
        mesh=_sc_mesh(),
        scratch_types=[pltpu.VMEM((4, GB), i32),
                       pltpu.VMEM((GB, D), f32), pltpu.VMEM((GB, D), f32),
                       pltpu.VMEM((GB, D), f32), pltpu.VMEM((GB, D), f32),
                       pltpu.SemaphoreType.DMA, pltpu.SemaphoreType.DMA,
                       pltpu.SemaphoreType.DMA, pltpu.SemaphoreType.DMA,
                       pltpu.SemaphoreType.DMA, pltpu.SemaphoreType.DMA,
                       pltpu.SemaphoreType.DMA, pltpu.SemaphoreType.DMA],
    )(idx_all, hidden, rela_embed, tpw, qr_table)


def _dense_body(rel_r, tpw_r, hs_r, hqr_r, head_r, tail_r,
                W1a_r, W2_r, b2_r,
                Wg1_r, Wg2_r, Wg3_r, Wg4_r, Wg5_r, bg_r,
                Wt1_r, Wt2_r, bt_r,
                Ws_r, Wr_r, Wq1_r, Wq2_r, Wq3_r, bqr_r, wa_r,
                up_r):
    lr = lambda x: jnp.where(x > 0, x, 0.01 * x)
    dot = lambda a, b: jnp.dot(a, b, preferred_element_type=jnp.float32)
    rel = rel_r[...]
    hs = hs_r[...]
    hqr = hqr_r[...]
    head = head_r[...]
    tail = tail_r[...]
    h1 = lr(dot(rel, W1a_r[...]) + tpw_r[...])
    h2 = lr(dot(h1, W2_r[...]) + b2_r[...])
    hr = h2 + rel
    gin = (dot(hr, Wg1_r[...])
           + 0.25 * (dot(hqr, Wg2_r[...]) + dot(head, Wg3_r[...])
                     + dot(tail, Wg4_r[...]))
           + dot(hs, Wg5_r[...]) + bg_r[...])
    gates = jax.nn.sigmoid(gin)
    update = gates[:, :D]
    reset = gates[:, D:]
    cand = jnp.tanh(dot(hr, Wt1_r[...]) + dot(reset * hs, Wt2_r[...]) + bt_r[...])
    message = (1.0 - update) * hs + update * cand
    att = lr(dot(hs, Ws_r[...]) + dot(hr, Wr_r[...]) + dot(hqr, Wq1_r[...])
             + dot(head, Wq2_r[...]) + dot(tail, Wq3_r[...]) + bqr_r[...])
    alpha = dot(att, wa_r[...])
    up_r[...] = jax.nn.sigmoid(alpha) * message


def _dense_stage(rel, tpw_g, hs, hqr, head, tail, weights):
    ce = rel.shape[0]
    nblk = (ce + BLK - 1) // BLK
    row_spec = lambda w: pl.BlockSpec((BLK, w), lambda i: (i, 0))
    full = lambda a: pl.BlockSpec(a.shape, lambda i: (0,) * a.ndim)
    return pl.pallas_call(
        _dense_body,
        grid=(nblk,),
        in_specs=[row_spec(D), row_spec(D), row_spec(D), row_spec(D),
                  row_spec(D), row_spec(D)] + [full(w) for w in weights],
        out_specs=row_spec(D),
        out_shape=jax.ShapeDtypeStruct((ce, D), jnp.float32),
    )(rel, tpw_g, hs, hqr, head, tail, *weights)


def _scatter_body(nblks, obj1_h, up1_h, obj2_h, up2_h, ones_h, ipu_h, ipd_h,
                  pu_o, pd_o,
                  obj_v, up_v, acc):
    cid = lax.axis_index("c")
    sid = lax.axis_index("s")
    r0 = sid * NPT

    @pl.when(cid == 0)
    def _init_up():
        pltpu.sync_copy(ipu_h.at[pl.ds(r0, NPT)], acc.at[pl.ds(r0, NPT)])

    @pl.when(cid == 1)
    def _init_deg():
        pltpu.sync_copy(ipd_h.at[pl.ds(r0, NPT)], acc.at[pl.ds(r0, NPT)])

    plsc.subcore_barrier()

    @pl.when(cid == 0)
    def _up_core():
        for nb, obj_h, up_h in zip(nblks, (obj1_h, obj2_h), (up1_h, up2_h)):
            def body(i, carry):
                base = (sid + i * NS) * GB
                pltpu.sync_copy(obj_h.at[pl.ds(base, GB)], obj_v)
                pltpu.sync_copy(up_h.at[pl.ds(base, GB)], up_v)
                pltpu.sync_copy(up_v, acc.at[obj_v], add=True)
                return carry

            lax.fori_loop(0, _share(nb, NS, sid), body, 0)

    @pl.when(cid == 1)
    def _deg_core():
        pltpu.sync_copy(ones_h, up_v)
        for nb, obj_h in zip(nblks, (obj1_h, obj2_h)):
            def body(i, carry):
                base = (sid + i * NS) * GB
                pltpu.sync_copy(obj_h.at[pl.ds(base, GB)], obj_v)
                pltpu.sync_copy(up_v, acc.at[obj_v], add=True)
                return carry

            lax.fori_loop(0, _share(nb, NS, sid), body, 0)

    plsc.subcore_barrier()

    @pl.when(cid == 0)
    def _out_up():
        pltpu.sync_copy(acc.at[pl.ds(r0, NPT)], pu_o.at[pl.ds(r0, NPT)])

    @pl.when(cid == 1)
    def _out_deg():
        pltpu.sync_copy(acc.at[pl.ds(r0, NPT)], pd_o.at[pl.ds(r0, NPT)])


def _scatter_stage(obj1, up1, obj2, up2, init_pu, init_pd):
    f32 = jnp.float32
    ones = jnp.ones((GB, D), f32)
    nbs = (obj1.shape[0] // GB, obj2.shape[0] // GB)
    return pl.kernel(
        functools.partial(_scatter_body, nbs),
        out_type=[jax.ShapeDtypeStruct((NP, D), f32),
                  jax.ShapeDtypeStruct((NP, D), f32)],
        mesh=_sc_mesh(),
        scratch_types=[pltpu.VMEM((GB,), jnp.int32),
                       pltpu.VMEM((GB, D), f32),
                       pltpu.VMEM_SHARED((NP, D), f32)],
    )(obj1, up1, obj2, up2, ones, init_pu, init_pd)


def _final_body(pu_r, pd_r, Wh_r, out_r):
    deg = pd_r[:, 0:1]
    agg = pu_r[...] / jnp.sqrt(deg + 0.0001)
    out_r[...] = jnp.dot(agg, Wh_r[...], preferred_element_type=jnp.float32)


def _final_stage(part_up, part_deg, Wh):
    rb = 2000
    return pl.pallas_call(
        _final_body,
        grid=(N // rb,),
        in_specs=[pl.BlockSpec((rb, D), lambda i: (i, 0)),
                  pl.BlockSpec((rb, D), lambda i: (i, 0)),
                  pl.BlockSpec(Wh.shape, lambda i: (0, 0))],
        out_specs=pl.BlockSpec((rb, D), lambda i: (i, 0)),
        out_shape=jax.ShapeDtypeStruct((N, D), jnp.float32),
    )(part_up, part_deg, Wh)


def kernel(q_sub, q_rel, hidden, edges, n_node, edge_head_rc_repr,
           edge_tail_rc_repr, query_head_rc_repr, rela_embed, time_pe,
           Ws, Wr, W1f, b1f, W2f, b2f, Wqr, bqr, w_alpha, Wg, bg,
           Wt, bt, Wh):
    f32 = jnp.float32
    sub = edges[:, 4].astype(jnp.int32)
    obj = edges[:, 5].astype(jnp.int32)
    r_idx = edges[:, 0].astype(jnp.int32)
    e2 = edges[:, 2].astype(jnp.int32)
    e6 = edges[:, 6].astype(jnp.int32)
    obj = obj + (jnp.asarray(n_node, dtype=obj.dtype) - N)

    q_rel_pad = jnp.pad(q_rel.astype(jnp.int32), (0, NQP - q_rel.shape[0]))
    qr_table = _qr_stage(q_rel_pad, rela_embed)
    tpw = _tpw_stage(time_pe, W1f[D:], b1f.reshape(1, D))

    weights = (
        W1f[:D], W2f, b2f.reshape(1, D),
        Wg[:D], Wg[D:2 * D], Wg[2 * D:3 * D], Wg[3 * D:4 * D], Wg[4 * D:],
        bg.reshape(1, 2 * D),
        Wt[:D], Wt[D:], bt.reshape(1, D),
        Ws, Wr, Wqr[:D], Wqr[D:2 * D], Wqr[2 * D:], bqr.reshape(1, D),
        w_alpha,
    )

    pu = jnp.zeros((NP, D), f32)
    pd = jnp.zeros((NP, D), f32)
    ups, objs = [], []
    for lo in range(0, E, CHUNK):
        hi = min(lo + CHUNK, E)
        nb = (hi - lo) // GB
        idx_all = jnp.stack([sub[lo:hi].reshape(nb, GB),
                             e2[lo:hi].reshape(nb, GB),
                             e6[lo:hi].reshape(nb, GB),
                             r_idx[lo:hi].reshape(nb, GB)], axis=1)
        hs_c, rel_c, tpw_c, hqr_c = _gather_stage(
            idx_all, hidden, rela_embed, tpw, qr_table)
        up_c = _dense_stage(rel_c, tpw_c, hs_c, hqr_c,
                            edge_head_rc_repr[lo:hi],
                            edge_tail_rc_repr[lo:hi], weights)
        objs.append(obj[lo:hi])
        ups.append(up_c)
        if len(ups) == 2:
            pu, pd = _scatter_stage(objs[0], ups[0], objs[1], ups[1], pu, pd)
            ups, objs = [], []

    return _final_stage(pu, pd, Wh)


# speedup vs baseline: 1.4874x; 1.0084x over previous
import functools

import jax
import jax.numpy as jnp
from jax import lax
from jax.experimental import pallas as pl
from jax.experimental.pallas import tpu as pltpu
from jax.experimental.pallas import tpu_sc as plsc

E = 160000
D = 128
TD = 32
N = 10000
NQP = 10240
BLK = 2048
GB = 128
NC, NS = 2, 16
NW = NC * NS
NP = 10240
NPT = NP // NS
CHUNK = 40960


@functools.lru_cache(maxsize=1)
def _sc_mesh():
    return plsc.VectorSubcoreMesh(
        core_axis_name="c", subcore_axis_name="s",
        num_cores=NC, num_subcores=NS)


def _share(total, workers, w):
    return total // workers + jnp.where(w < total % workers, 1, 0)


def _qr_body(qrel_h, rela_h, qrt_o, idx_v, rows_v):
    cid = lax.axis_index("c")
    sid = lax.axis_index("s")
    wid = sid * NC + cid

    def body(i, carry):
        base = (wid + i * NW) * GB
        pltpu.sync_copy(qrel_h.at[pl.ds(base, GB)], idx_v)
        pltpu.sync_copy(rela_h.at[idx_v], rows_v)
        pltpu.sync_copy(rows_v, qrt_o.at[pl.ds(base, GB)])
        return carry

    lax.fori_loop(0, _share(NQP // GB, NW, wid), body, 0)


def _qr_stage(q_rel_pad, rela_embed):
    return pl.kernel(
        _qr_body,
        out_type=jax.ShapeDtypeStruct((NQP, D), jnp.float32),
        mesh=_sc_mesh(),
        scratch_types=[pltpu.VMEM((GB,), jnp.int32),
                       pltpu.VMEM((GB, D), jnp.float32)],
    )(q_rel_pad, rela_embed)


def _tpw_body(tp_r, W1b_r, b1_r, out_r):
    out_r[...] = (jnp.dot(tp_r[...], W1b_r[...],
                          preferred_element_type=jnp.float32) + b1_r[...])


def _tpw_stage(time_pe, W1b, b1f):
    rb = 2000
    return pl.pallas_call(
        _tpw_body,
        grid=(N // rb,),
        in_specs=[pl.BlockSpec((rb, TD), lambda i: (i, 0)),
                  pl.BlockSpec((TD, D), lambda i: (0, 0)),
                  pl.BlockSpec((1, D), lambda i: (0, 0))],
        out_specs=pl.BlockSpec((rb, D), lambda i: (i, 0)),
        out_shape=jax.ShapeDtypeStruct((N, D), jnp.float32),
    )(time_pe, W1b, b1f)


GG = 160


def _gather_body(nblk_total, idx_h, hidden_h, rela_h, tpw_h, qrt_h,
                 hs_o, rel_o, tpw_o, hqr_o,
                 idx_v, hs_v, rel_v, tpw_v, hqr_v,
                 s0, s1, s2, s3, t0, t1, t2, t3):
    cid = lax.axis_index("c")
    sid = lax.axis_index("s")
    wid = sid * NC + cid
    tables = (hidden_h, rela_h, tpw_h, qrt_h)
    bufs = (hs_v, rel_v, tpw_v, hqr_v)
    gsems = (s0, s1, s2, s3)
    wsems = (t0, t1, t2, t3)
    outs = (hs_o, rel_o, tpw_o, hqr_o)

    def body(i, carry):
        b = wid + i * NW
        base = b * GG
        pltpu.sync_copy(idx_h.at[b], idx_v)
        gcs = []
        for j in range(4):
            gcs.append(pltpu.async_copy(
                tables[j].at[idx_v.at[j, pl.ds(0, 128)]],
                bufs[j].at[pl.ds(0, 128)], gsems[j]))
            gcs.append(pltpu.async_copy(
                tables[j].at[idx_v.at[j, pl.ds(128, GG - 128)]],
                bufs[j].at[pl.ds(128, GG - 128)], gsems[j]))
        wcs = []
        for j in range(4):
            gcs[2 * j].wait()
            gcs[2 * j + 1].wait()
            wcs.append(pltpu.async_copy(bufs[j], outs[j].at[pl.ds(base, GG)],
                                        wsems[j]))
        for w in wcs:
            w.wait()
        return carry

    lax.fori_loop(0, _share(nblk_total, NW, wid), body, 0)


def _gather_stage(idx_all, hidden, rela_embed, tpw, qr_table):
    f32 = jnp.float32
    i32 = jnp.int32
    nb = idx_all.shape[0]
    ce = nb * GG
    return pl.kernel(
        functools.partial(_gather_body, nb),
        out_type=[jax.ShapeDtypeStruct((ce, D), f32),
                  jax.ShapeDtypeStruct((ce, D), f32),
                  jax.ShapeDtypeStruct((ce, D), f32),
                  jax.ShapeDtypeStruct((ce, D), f32)],
        mesh=_sc_mesh(),
        scratch_types=[pltpu.VMEM((4, GG), i32),
                       pltpu.VMEM((GG, D), f32), pltpu.VMEM((GG, D), f32),
                       pltpu.VMEM((GG, D), f32), pltpu.VMEM((GG, D), f32),
                       pltpu.SemaphoreType.DMA, pltpu.SemaphoreType.DMA,
                       pltpu.SemaphoreType.DMA, pltpu.SemaphoreType.DMA,
                       pltpu.SemaphoreType.DMA, pltpu.SemaphoreType.DMA,
                       pltpu.SemaphoreType.DMA, pltpu.SemaphoreType.DMA],
    )(idx_all, hidden, rela_embed, tpw, qr_table)


def _dense_body(rel_r, tpw_r, hs_r, hqr_r, head_r, tail_r,
                W1a_r, W2_r, b2_r,
                Wg1_r, Wg2_r, Wg3_r, Wg4_r, Wg5_r, bg_r,
                Wt1_r, Wt2_r, bt_r,
                Ws_r, Wr_r, Wq1_r, Wq2_r, Wq3_r, bqr_r, wa_r,
                up_r):
    lr = lambda x: jnp.where(x > 0, x, 0.01 * x)
    dot = lambda a, b: jnp.dot(a, b, preferred_element_type=jnp.float32)
    rel = rel_r[...]
    hs = hs_r[...]
    hqr = hqr_r[...]
    head = head_r[...]
    tail = tail_r[...]
    h1 = lr(dot(rel, W1a_r[...]) + tpw_r[...])
    h2 = lr(dot(h1, W2_r[...]) + b2_r[...])
    hr = h2 + rel
    gin = (dot(hr, Wg1_r[...])
           + 0.25 * (dot(hqr, Wg2_r[...]) + dot(head, Wg3_r[...])
                     + dot(tail, Wg4_r[...]))
           + dot(hs, Wg5_r[...]) + bg_r[...])
    gates = jax.nn.sigmoid(gin)
    update = gates[:, :D]
    reset = gates[:, D:]
    cand = jnp.tanh(dot(hr, Wt1_r[...]) + dot(reset * hs, Wt2_r[...]) + bt_r[...])
    message = (1.0 - update) * hs + update * cand
    att = lr(dot(hs, Ws_r[...]) + dot(hr, Wr_r[...]) + dot(hqr, Wq1_r[...])
             + dot(head, Wq2_r[...]) + dot(tail, Wq3_r[...]) + bqr_r[...])
    alpha = dot(att, wa_r[...])
    up_r[...] = jax.nn.sigmoid(alpha) * message


def _dense_stage(rel, tpw_g, hs, hqr, head, tail, weights):
    ce = rel.shape[0]
    nblk = (ce + BLK - 1) // BLK
    row_spec = lambda w: pl.BlockSpec((BLK, w), lambda i: (i, 0))
    full = lambda a: pl.BlockSpec(a.shape, lambda i: (0,) * a.ndim)
    return pl.pallas_call(
        _dense_body,
        grid=(nblk,),
        in_specs=[row_spec(D), row_spec(D), row_spec(D), row_spec(D),
                  row_spec(D), row_spec(D)] + [full(w) for w in weights],
        out_specs=row_spec(D),
        out_shape=jax.ShapeDtypeStruct((ce, D), jnp.float32),
    )(rel, tpw_g, hs, hqr, head, tail, *weights)


def _scatter_body(nblks, obj1_h, up1_h, obj2_h, up2_h, ones_h, ipu_h, ipd_h,
                  pu_o, pd_o,
                  obj_v, up_v, acc, usem):
    cid = lax.axis_index("c")
    sid = lax.axis_index("s")
    r0 = sid * NPT

    @pl.when(cid == 0)
    def _init_up():
        pltpu.sync_copy(ipu_h.at[pl.ds(r0, NPT)], acc.at[pl.ds(r0, NPT)])

    @pl.when(cid == 1)
    def _init_deg():
        pltpu.sync_copy(ipd_h.at[pl.ds(r0, NPT)], acc.at[pl.ds(r0, NPT)])

    plsc.subcore_barrier()

    @pl.when(cid == 0)
    def _up_core():
        for nb, obj_h, up_h in zip(nblks, (obj1_h, obj2_h), (up1_h, up2_h)):
            def body(i, carry):
                base = (sid + i * NS) * GB
                c = pltpu.async_copy(up_h.at[pl.ds(base, GB)], up_v, usem)
                pltpu.sync_copy(obj_h.at[pl.ds(base, GB)], obj_v)
                c.wait()
                pltpu.sync_copy(up_v, acc.at[obj_v], add=True)
                return carry

            lax.fori_loop(0, _share(nb, NS, sid), body, 0)

    @pl.when(cid == 1)
    def _deg_core():
        pltpu.sync_copy(ones_h, up_v)
        for nb, obj_h in zip(nblks, (obj1_h, obj2_h)):
            def body(i, carry):
                base = (sid + i * NS) * GB
                pltpu.sync_copy(obj_h.at[pl.ds(base, GB)], obj_v)
                pltpu.sync_copy(up_v, acc.at[obj_v], add=True)
                return carry

            lax.fori_loop(0, _share(nb, NS, sid), body, 0)

    plsc.subcore_barrier()

    @pl.when(cid == 0)
    def _out_up():
        pltpu.sync_copy(acc.at[pl.ds(r0, NPT)], pu_o.at[pl.ds(r0, NPT)])

    @pl.when(cid == 1)
    def _out_deg():
        pltpu.sync_copy(acc.at[pl.ds(r0, NPT)], pd_o.at[pl.ds(r0, NPT)])


def _scatter_stage(obj1, up1, obj2, up2, init_pu, init_pd):
    f32 = jnp.float32
    ones = jnp.ones((GB, D), f32)
    nbs = (obj1.shape[0] // GB, obj2.shape[0] // GB)
    return pl.kernel(
        functools.partial(_scatter_body, nbs),
        out_type=[jax.ShapeDtypeStruct((NP, D), f32),
                  jax.ShapeDtypeStruct((NP, D), f32)],
        mesh=_sc_mesh(),
        scratch_types=[pltpu.VMEM((GB,), jnp.int32),
                       pltpu.VMEM((GB, D), f32),
                       pltpu.VMEM_SHARED((NP, D), f32),
                       pltpu.SemaphoreType.DMA],
    )(obj1, up1, obj2, up2, ones, init_pu, init_pd)


def _final_body(pu_r, pd_r, Wh_r, out_r):
    deg = pd_r[:, 0:1]
    agg = pu_r[...] / jnp.sqrt(deg + 0.0001)
    out_r[...] = jnp.dot(agg, Wh_r[...], preferred_element_type=jnp.float32)


def _final_stage(part_up, part_deg, Wh):
    rb = 2000
    return pl.pallas_call(
        _final_body,
        grid=(N // rb,),
        in_specs=[pl.BlockSpec((rb, D), lambda i: (i, 0)),
                  pl.BlockSpec((rb, D), lambda i: (i, 0)),
                  pl.BlockSpec(Wh.shape, lambda i: (0, 0))],
        out_specs=pl.BlockSpec((rb, D), lambda i: (i, 0)),
        out_shape=jax.ShapeDtypeStruct((N, D), jnp.float32),
    )(part_up, part_deg, Wh)


def kernel(q_sub, q_rel, hidden, edges, n_node, edge_head_rc_repr,
           edge_tail_rc_repr, query_head_rc_repr, rela_embed, time_pe,
           Ws, Wr, W1f, b1f, W2f, b2f, Wqr, bqr, w_alpha, Wg, bg,
           Wt, bt, Wh):
    f32 = jnp.float32
    sub = edges[:, 4].astype(jnp.int32)
    obj = edges[:, 5].astype(jnp.int32)
    r_idx = edges[:, 0].astype(jnp.int32)
    e2 = edges[:, 2].astype(jnp.int32)
    e6 = edges[:, 6].astype(jnp.int32)
    obj = obj + (jnp.asarray(n_node, dtype=obj.dtype) - N)

    q_rel_pad = jnp.pad(q_rel.astype(jnp.int32), (0, NQP - q_rel.shape[0]))
    qr_table = _qr_stage(q_rel_pad, rela_embed)
    tpw = _tpw_stage(time_pe, W1f[D:], b1f.reshape(1, D))

    weights = (
        W1f[:D], W2f, b2f.reshape(1, D),
        Wg[:D], Wg[D:2 * D], Wg[2 * D:3 * D], Wg[3 * D:4 * D], Wg[4 * D:],
        bg.reshape(1, 2 * D),
        Wt[:D], Wt[D:], bt.reshape(1, D),
        Ws, Wr, Wqr[:D], Wqr[D:2 * D], Wqr[2 * D:], bqr.reshape(1, D),
        w_alpha,
    )

    pu = jnp.zeros((NP, D), f32)
    pd = jnp.zeros((NP, D), f32)
    ups, objs = [], []
    for lo in range(0, E, CHUNK):
        hi = min(lo + CHUNK, E)
        nb = (hi - lo) // GG
        idx_all = jnp.stack([sub[lo:hi].reshape(nb, GG),
                             e2[lo:hi].reshape(nb, GG),
                             e6[lo:hi].reshape(nb, GG),
                             r_idx[lo:hi].reshape(nb, GG)], axis=1)
        hs_c, rel_c, tpw_c, hqr_c = _gather_stage(
            idx_all, hidden, rela_embed, tpw, qr_table)
        up_c = _dense_stage(rel_c, tpw_c, hs_c, hqr_c,
                            edge_head_rc_repr[lo:hi],
                            edge_tail_rc_repr[lo:hi], weights)
        objs.append(obj[lo:hi])
        ups.append(up_c)
        if len(ups) == 2:
            pu, pd = _scatter_stage(objs[0], ups[0], objs[1], ups[1], pu, pd)
            ups, objs = [], []

    return _final_stage(pu, pd, Wh)


# speedup vs baseline: 1.5015x; 1.0095x over previous
import functools

import jax
import jax.numpy as jnp
from jax import lax
from jax.experimental import pallas as pl
from jax.experimental.pallas import tpu as pltpu
from jax.experimental.pallas import tpu_sc as plsc

E = 160000
D = 128
TD = 32
N = 10000
NQP = 10240
BLK = 2048
GB = 128
NC, NS = 2, 16
NW = NC * NS
NP = 10240
NPT = NP // NS
CHUNK = 32000


@functools.lru_cache(maxsize=1)
def _sc_mesh():
    return plsc.VectorSubcoreMesh(
        core_axis_name="c", subcore_axis_name="s",
        num_cores=NC, num_subcores=NS)


def _share(total, workers, w):
    return total // workers + jnp.where(w < total % workers, 1, 0)


def _qr_body(qrel_h, rela_h, qrt_o, idx_v, rows_v):
    cid = lax.axis_index("c")
    sid = lax.axis_index("s")
    wid = sid * NC + cid

    def body(i, carry):
        base = (wid + i * NW) * GB
        pltpu.sync_copy(qrel_h.at[pl.ds(base, GB)], idx_v)
        pltpu.sync_copy(rela_h.at[idx_v], rows_v)
        pltpu.sync_copy(rows_v, qrt_o.at[pl.ds(base, GB)])
        return carry

    lax.fori_loop(0, _share(NQP // GB, NW, wid), body, 0)


def _qr_stage(q_rel_pad, rela_embed):
    return pl.kernel(
        _qr_body,
        out_type=jax.ShapeDtypeStruct((NQP, D), jnp.float32),
        mesh=_sc_mesh(),
        scratch_types=[pltpu.VMEM((GB,), jnp.int32),
                       pltpu.VMEM((GB, D), jnp.float32)],
    )(q_rel_pad, rela_embed)


def _tpw_body(tp_r, W1b_r, b1_r, out_r):
    out_r[...] = (jnp.dot(tp_r[...], W1b_r[...],
                          preferred_element_type=jnp.float32) + b1_r[...])


def _tpw_stage(time_pe, W1b, b1f):
    rb = 2000
    return pl.pallas_call(
        _tpw_body,
        grid=(N // rb,),
        in_specs=[pl.BlockSpec((rb, TD), lambda i: (i, 0)),
                  pl.BlockSpec((TD, D), lambda i: (0, 0)),
                  pl.BlockSpec((1, D), lambda i: (0, 0))],
        out_specs=pl.BlockSpec((rb, D), lambda i: (i, 0)),
        out_shape=jax.ShapeDtypeStruct((N, D), jnp.float32),
    )(time_pe, W1b, b1f)


GG = 160


def _gather_body(nblk_total, idx_h, hidden_h, rela_h, tpw_h, qrt_h,
                 hs_o, rel_o, tpw_o, hqr_o,
                 idx_v, hs_v, rel_v, tpw_v, hqr_v,
                 s0, s1, s2, s3, t0, t1, t2, t3):
    cid = lax.axis_index("c")
    sid = lax.axis_index("s")
    wid = sid * NC + cid
    tables = (hidden_h, rela_h, tpw_h, qrt_h)
    bufs = (hs_v, rel_v, tpw_v, hqr_v)
    gsems = (s0, s1, s2, s3)
    wsems = (t0, t1, t2, t3)
    outs = (hs_o, rel_o, tpw_o, hqr_o)

    def body(i, carry):
        b = wid + i * NW
        base = b * GG
        pltpu.sync_copy(idx_h.at[b], idx_v)
        gcs = []
        for j in range(4):
            gcs.append(pltpu.async_copy(
                tables[j].at[idx_v.at[j, pl.ds(0, 128)]],
                bufs[j].at[pl.ds(0, 128)], gsems[j]))
            gcs.append(pltpu.async_copy(
                tables[j].at[idx_v.at[j, pl.ds(128, GG - 128)]],
                bufs[j].at[pl.ds(128, GG - 128)], gsems[j]))
        wcs = []
        for j in range(4):
            gcs[2 * j].wait()
            gcs[2 * j + 1].wait()
            wcs.append(pltpu.async_copy(bufs[j], outs[j].at[pl.ds(base, GG)],
                                        wsems[j]))
        for w in wcs:
            w.wait()
        return carry

    lax.fori_loop(0, _share(nblk_total, NW, wid), body, 0)


def _gather_stage(idx_all, hidden, rela_embed, tpw, qr_table):
    f32 = jnp.float32
    i32 = jnp.int32
    nb = idx_all.shape[0]
    ce = nb * GG
    return pl.kernel(
        functools.partial(_gather_body, nb),
        out_type=[jax.ShapeDtypeStruct((ce, D), f32),
                  jax.ShapeDtypeStruct((ce, D), f32),
                  jax.ShapeDtypeStruct((ce, D), f32),
                  jax.ShapeDtypeStruct((ce, D), f32)],
        mesh=_sc_mesh(),
        scratch_types=[pltpu.VMEM((4, GG), i32),
                       pltpu.VMEM((GG, D), f32), pltpu.VMEM((GG, D), f32),
                       pltpu.VMEM((GG, D), f32), pltpu.VMEM((GG, D), f32),
                       pltpu.SemaphoreType.DMA, pltpu.SemaphoreType.DMA,
                       pltpu.SemaphoreType.DMA, pltpu.SemaphoreType.DMA,
                       pltpu.SemaphoreType.DMA, pltpu.SemaphoreType.DMA,
                       pltpu.SemaphoreType.DMA, pltpu.SemaphoreType.DMA],
    )(idx_all, hidden, rela_embed, tpw, qr_table)


def _dense_body(rel_r, tpw_r, hs_r, hqr_r, head_r, tail_r,
                W1a_r, W2_r, b2_r,
                Wg1_r, Wg2_r, Wg3_r, Wg4_r, Wg5_r, bg_r,
                Wt1_r, Wt2_r, bt_r,
                Ws_r, Wr_r, Wq1_r, Wq2_r, Wq3_r, bqr_r, wa_r,
                up_r):
    lr = lambda x: jnp.where(x > 0, x, 0.01 * x)
    dot = lambda a, b: jnp.dot(a, b, preferred_element_type=jnp.float32)
    rel = rel_r[...]
    hs = hs_r[...]
    hqr = hqr_r[...]
    head = head_r[...]
    tail = tail_r[...]
    h1 = lr(dot(rel, W1a_r[...]) + tpw_r[...])
    h2 = lr(dot(h1, W2_r[...]) + b2_r[...])
    hr = h2 + rel
    gin = (dot(hr, Wg1_r[...])
           + 0.25 * (dot(hqr, Wg2_r[...]) + dot(head, Wg3_r[...])
                     + dot(tail, Wg4_r[...]))
           + dot(hs, Wg5_r[...]) + bg_r[...])
    gates = jax.nn.sigmoid(gin)
    update = gates[:, :D]
    reset = gates[:, D:]
    cand = jnp.tanh(dot(hr, Wt1_r[...]) + dot(reset * hs, Wt2_r[...]) + bt_r[...])
    message = (1.0 - update) * hs + update * cand
    att = lr(dot(hs, Ws_r[...]) + dot(hr, Wr_r[...]) + dot(hqr, Wq1_r[...])
             + dot(head, Wq2_r[...]) + dot(tail, Wq3_r[...]) + bqr_r[...])
    alpha = dot(att, wa_r[...])
    up_r[...] = jax.nn.sigmoid(alpha) * message


def _dense_stage(rel, tpw_g, hs, hqr, head, tail, weights):
    ce = rel.shape[0]
    nblk = (ce + BLK - 1) // BLK
    row_spec = lambda w: pl.BlockSpec((BLK, w), lambda i: (i, 0))
    full = lambda a: pl.BlockSpec(a.shape, lambda i: (0,) * a.ndim)
    return pl.pallas_call(
        _dense_body,
        grid=(nblk,),
        in_specs=[row_spec(D), row_spec(D), row_spec(D), row_spec(D),
                  row_spec(D), row_spec(D)] + [full(w) for w in weights],
        out_specs=row_spec(D),
        out_shape=jax.ShapeDtypeStruct((ce, D), jnp.float32),
    )(rel, tpw_g, hs, hqr, head, tail, *weights)


def _scatter_body(nblks, *args):
    k = len(nblks)
    obj_hs = args[0:k]
    up_hs = args[k:2 * k]
    (ones_h, ipu_h, ipd_h, pu_o, pd_o, obj_v, up_v, acc, usem) = args[2 * k:]
    cid = lax.axis_index("c")
    sid = lax.axis_index("s")
    r0 = sid * NPT

    @pl.when(cid == 0)
    def _init_up():
        pltpu.sync_copy(ipu_h.at[pl.ds(r0, NPT)], acc.at[pl.ds(r0, NPT)])

    @pl.when(cid == 1)
    def _init_deg():
        pltpu.sync_copy(ipd_h.at[pl.ds(r0, NPT)], acc.at[pl.ds(r0, NPT)])

    plsc.subcore_barrier()

    @pl.when(cid == 0)
    def _up_core():
        for nb, obj_h, up_h in zip(nblks, obj_hs, up_hs):
            def body(i, carry):
                base = (sid + i * NS) * GB
                c = pltpu.async_copy(up_h.at[pl.ds(base, GB)], up_v, usem)
                pltpu.sync_copy(obj_h.at[pl.ds(base, GB)], obj_v)
                c.wait()
                pltpu.sync_copy(up_v, acc.at[obj_v], add=True)
                return carry

            lax.fori_loop(0, _share(nb, NS, sid), body, 0)

    @pl.when(cid == 1)
    def _deg_core():
        pltpu.sync_copy(ones_h, up_v)
        for nb, obj_h in zip(nblks, obj_hs):
            def body(i, carry):
                base = (sid + i * NS) * GB
                pltpu.sync_copy(obj_h.at[pl.ds(base, GB)], obj_v)
                pltpu.sync_copy(up_v, acc.at[obj_v], add=True)
                return carry

            lax.fori_loop(0, _share(nb, NS, sid), body, 0)

    plsc.subcore_barrier()

    @pl.when(cid == 0)
    def _out_up():
        pltpu.sync_copy(acc.at[pl.ds(r0, NPT)], pu_o.at[pl.ds(r0, NPT)])

    @pl.when(cid == 1)
    def _out_deg():
        pltpu.sync_copy(acc.at[pl.ds(r0, NPT)], pd_o.at[pl.ds(r0, NPT)])


def _scatter_stage(pairs, init_pu, init_pd):
    f32 = jnp.float32
    ones = jnp.ones((GB, D), f32)
    objs = [p[0] for p in pairs]
    ups = [p[1] for p in pairs]
    nbs = tuple(o.shape[0] // GB for o in objs)
    return pl.kernel(
        functools.partial(_scatter_body, nbs),
        out_type=[jax.ShapeDtypeStruct((NP, D), f32),
                  jax.ShapeDtypeStruct((NP, D), f32)],
        mesh=_sc_mesh(),
        scratch_types=[pltpu.VMEM((GB,), jnp.int32),
                       pltpu.VMEM((GB, D), f32),
                       pltpu.VMEM_SHARED((NP, D), f32),
                       pltpu.SemaphoreType.DMA],
    )(*objs, *ups, ones, init_pu, init_pd)


def _final_body(pu_r, pd_r, Wh_r, out_r):
    deg = pd_r[:, 0:1]
    agg = pu_r[...] / jnp.sqrt(deg + 0.0001)
    out_r[...] = jnp.dot(agg, Wh_r[...], preferred_element_type=jnp.float32)


def _final_stage(part_up, part_deg, Wh):
    rb = 2000
    return pl.pallas_call(
        _final_body,
        grid=(N // rb,),
        in_specs=[pl.BlockSpec((rb, D), lambda i: (i, 0)),
                  pl.BlockSpec((rb, D), lambda i: (i, 0)),
                  pl.BlockSpec(Wh.shape, lambda i: (0, 0))],
        out_specs=pl.BlockSpec((rb, D), lambda i: (i, 0)),
        out_shape=jax.ShapeDtypeStruct((N, D), jnp.float32),
    )(part_up, part_deg, Wh)


def kernel(q_sub, q_rel, hidden, edges, n_node, edge_head_rc_repr,
           edge_tail_rc_repr, query_head_rc_repr, rela_embed, time_pe,
           Ws, Wr, W1f, b1f, W2f, b2f, Wqr, bqr, w_alpha, Wg, bg,
           Wt, bt, Wh):
    f32 = jnp.float32
    sub = edges[:, 4].astype(jnp.int32)
    obj = edges[:, 5].astype(jnp.int32)
    r_idx = edges[:, 0].astype(jnp.int32)
    e2 = edges[:, 2].astype(jnp.int32)
    e6 = edges[:, 6].astype(jnp.int32)
    obj = obj + (jnp.asarray(n_node, dtype=obj.dtype) - N)

    q_rel_pad = jnp.pad(q_rel.astype(jnp.int32), (0, NQP - q_rel.shape[0]))
    qr_table = _qr_stage(q_rel_pad, rela_embed)
    tpw = _tpw_stage(time_pe, W1f[D:], b1f.reshape(1, D))

    weights = (
        W1f[:D], W2f, b2f.reshape(1, D),
        Wg[:D], Wg[D:2 * D], Wg[2 * D:3 * D], Wg[3 * D:4 * D], Wg[4 * D:],
        bg.reshape(1, 2 * D),
        Wt[:D], Wt[D:], bt.reshape(1, D),
        Ws, Wr, Wqr[:D], Wqr[D:2 * D], Wqr[2 * D:], bqr.reshape(1, D),
        w_alpha,
    )

    pu = jnp.zeros((NP, D), f32)
    pd = jnp.zeros((NP, D), f32)
    pending = []
    for lo in range(0, E, CHUNK):
        hi = min(lo + CHUNK, E)
        nb = (hi - lo) // GG
        idx_all = jnp.stack([sub[lo:hi].reshape(nb, GG),
                             e2[lo:hi].reshape(nb, GG),
                             e6[lo:hi].reshape(nb, GG),
                             r_idx[lo:hi].reshape(nb, GG)], axis=1)
        hs_c, rel_c, tpw_c, hqr_c = _gather_stage(
            idx_all, hidden, rela_embed, tpw, qr_table)
        up_c = _dense_stage(rel_c, tpw_c, hs_c, hqr_c,
                            edge_head_rc_repr[lo:hi],
                            edge_tail_rc_repr[lo:hi], weights)
        pending.append((obj[lo:hi], up_c))
        if len(pending) == 2:
            pu, pd = _scatter_stage(pending, pu, pd)
            pending = []
    if pending:
        pu, pd = _scatter_stage(pending, pu, pd)

    return _final_stage(pu, pd, Wh)
